# baseline (speedup 1.0000x reference)
import numpy as np
import jax
import jax.numpy as jnp
from jax.experimental import pallas as pl
from jax.experimental.pallas import tpu as pltpu

_B = 128
_V = 100000
_C = 6400
_NB = (_V + _C - 1) // _C

_TINY = np.float32(np.finfo(np.float32).tiny)


def _sampler_body(x_ref, cnt_ref, out_ref, bestv_ref, besti_ref):
    j = pl.program_id(0)

    @pl.when(j == 0)
    def _init():
        bestv_ref[...] = jnp.full((_B, 1), -jnp.inf, jnp.float32)
        besti_ref[...] = jnp.zeros((_B, 1), jnp.int32)

    x = x_ref[...]
    x1 = cnt_ref[...] + jnp.uint32(j * _C)

    ks = (jnp.uint32(0), jnp.uint32(1), jnp.uint32(0x1BD11BDB))
    rot = ((13, 15, 26, 6), (17, 29, 16, 24))
    x0 = jnp.zeros((_B, _C), jnp.uint32) + ks[0]
    for r in range(5):
        for rr in rot[r % 2]:
            x0 = x0 + x1
            x1 = (x1 << jnp.uint32(rr)) | (x1 >> jnp.uint32(32 - rr))
            x1 = x0 ^ x1
        x0 = x0 + ks[(r + 1) % 3]
        x1 = x1 + ks[(r + 2) % 3] + jnp.uint32(r + 1)
    bits = x0 ^ x1

    fb = (bits >> jnp.uint32(9)) | jnp.uint32(0x3F800000)
    f = jax.lax.bitcast_convert_type(fb, jnp.float32) - jnp.float32(1.0)
    u = jnp.maximum(f, _TINY)
    g = -jnp.log(-jnp.log(u))

    v = x + g
    col = jax.lax.broadcasted_iota(jnp.int32, (_B, _C), 1)
    v = jnp.where(col < _V - j * _C, v, -jnp.inf)

    m = jnp.max(v, axis=1, keepdims=True)
    cand = jnp.where(v == m, col, jnp.int32(0x7FFFFFFF))
    idx = jnp.min(cand, axis=1, keepdims=True) + j * _C

    upd = m > bestv_ref[...]
    bestv_ref[...] = jnp.where(upd, m, bestv_ref[...])
    besti_ref[...] = jnp.where(upd, idx, besti_ref[...])

    @pl.when(j == _NB - 1)
    def _fin():
        out_ref[...] = besti_ref[...]


def _base_counters():
    r = np.arange(_B, dtype=np.uint64)[:, None]
    c = np.arange(_C, dtype=np.uint64)[None, :]
    return jnp.asarray((r * _V + c + 1).astype(np.uint32))


def kernel(logits):
    cnt0 = _base_counters()
    out = pl.pallas_call(
        _sampler_body,
        grid=(_NB,),
        in_specs=[
            pl.BlockSpec((_B, _C), lambda j: (0, j)),
            pl.BlockSpec((_B, _C), lambda j: (0, 0)),
        ],
        out_specs=pl.BlockSpec((_B, 1), lambda j: (0, 0)),
        out_shape=jax.ShapeDtypeStruct((_B, 1), jnp.int32),
        scratch_shapes=[
            pltpu.VMEM((_B, 1), jnp.float32),
            pltpu.VMEM((_B, 1), jnp.int32),
        ],
    )(logits, cnt0)
    return out.reshape(_B)


# speedup vs baseline: 1.2924x; 1.2924x over previous
import numpy as np
import jax
import jax.numpy as jnp
from jax.experimental import pallas as pl
from jax.experimental.pallas import tpu as pltpu

_B = 128
_V = 100000
_C = 3584
_NB = (_V + _C - 1) // _C

_TINY = np.float32(np.finfo(np.float32).tiny)


def _sampler_body(x_ref, cnt_ref, out_ref, bestv_ref, besti_ref):
    j = pl.program_id(0)

    @pl.when(j == 0)
    def _init():
        bestv_ref[...] = jnp.full((_B, 1), -jnp.inf, jnp.float32)
        besti_ref[...] = jnp.zeros((_B, 1), jnp.int32)

    x = x_ref[...]
    x1 = cnt_ref[...] + jnp.uint32(j * _C)

    ks = (jnp.uint32(0), jnp.uint32(1), jnp.uint32(0x1BD11BDB))
    rot = ((13, 15, 26, 6), (17, 29, 16, 24))
    x0 = jnp.zeros((_B, _C), jnp.uint32) + ks[0]
    for r in range(5):
        for rr in rot[r % 2]:
            x0 = x0 + x1
            x1 = (x1 << jnp.uint32(rr)) | (x1 >> jnp.uint32(32 - rr))
            x1 = x0 ^ x1
        x0 = x0 + ks[(r + 1) % 3]
        x1 = x1 + ks[(r + 2) % 3] + jnp.uint32(r + 1)
    bits = x0 ^ x1

    fb = (bits >> jnp.uint32(9)) | jnp.uint32(0x3F800000)
    f = jax.lax.bitcast_convert_type(fb, jnp.float32) - jnp.float32(1.0)
    u = jnp.maximum(f, _TINY)
    g = -jnp.log(-jnp.log(u))

    v = x + g
    col = jax.lax.broadcasted_iota(jnp.int32, (_B, _C), 1)
    v = jnp.where(col < _V - j * _C, v, -jnp.inf)

    m = jnp.max(v, axis=1, keepdims=True)
    cand = jnp.where(v == m, col, jnp.int32(0x7FFFFFFF))
    idx = jnp.min(cand, axis=1, keepdims=True) + j * _C

    upd = m > bestv_ref[...]
    bestv_ref[...] = jnp.where(upd, m, bestv_ref[...])
    besti_ref[...] = jnp.where(upd, idx, besti_ref[...])

    @pl.when(j == _NB - 1)
    def _fin():
        out_ref[...] = besti_ref[...]


def _base_counters():
    r = np.arange(_B, dtype=np.uint64)[:, None]
    c = np.arange(_C, dtype=np.uint64)[None, :]
    return jnp.asarray((r * _V + c + 1).astype(np.uint32))


def kernel(logits):
    cnt0 = _base_counters()
    out = pl.pallas_call(
        _sampler_body,
        grid=(_NB,),
        in_specs=[
            pl.BlockSpec((_B, _C), lambda j: (0, j)),
            pl.BlockSpec((_B, _C), lambda j: (0, 0)),
        ],
        out_specs=pl.BlockSpec((_B, 1), lambda j: (0, 0)),
        out_shape=jax.ShapeDtypeStruct((_B, 1), jnp.int32),
        scratch_shapes=[
            pltpu.VMEM((_B, 1), jnp.float32),
            pltpu.VMEM((_B, 1), jnp.int32),
        ],
    )(logits, cnt0)
    return out.reshape(_B)
